# Initial kernel scaffold; baseline (speedup 1.0000x reference)
import functools

import jax
import jax.numpy as jnp
from jax.experimental import pallas as pl
from jax.experimental.pallas import tpu as pltpu

_K = 5


def _taps(a, w_stride, length):
    rows = [a]
    for kh in range(1, _K):
        rows.append(pltpu.roll(a, length - kh * w_stride, axis=1))
    stacked = jnp.concatenate(rows, axis=0)
    cols = [stacked]
    for kw in range(1, _K):
        cols.append(pltpu.roll(stacked, length - kw, axis=1))
    return jnp.concatenate(cols, axis=0)


def _fwd_kernel(x_ref, w1_ref, b1_ref, w2_ref, b2_ref, wav_ref,
                out_ref, feat_ref, *, s, n, w_stride, keep):
    length = s * n
    xs = x_ref[...]
    xc = jnp.concatenate([xs[i] for i in range(s)], axis=1)
    xc = xc.astype(jnp.bfloat16)

    t1 = _taps(xc, w_stride, length)
    h1 = jnp.dot(w1_ref[...], t1, preferred_element_type=jnp.float32)
    h1 = jnp.maximum(h1 + b1_ref[...], 0.0).astype(jnp.bfloat16)

    t2 = _taps(h1, w_stride, length)
    h2 = jnp.dot(w2_ref[...], t2, preferred_element_type=jnp.float32)
    h2 = jnp.maximum(h2 + b2_ref[...], 0.0)

    for i in range(s):
        feat_ref[i] = h2[:, i * n:i * n + keep]

    out_ref[...] = jnp.log(wav_ref[...]) * (1.0 / 20.0)


def kernel(w1, b1, w2, b2, x, wav):
    B, Cin, H, W = x.shape
    C1, C2 = w1.shape[0], w2.shape[0]
    H2, W2 = H - 2 * (_K - 1), W - 2 * (_K - 1)
    n = H * W
    keep = H2 * W
    Nw = wav.shape[-1]
    S = 4
    assert B % S == 0

    x_flat = x.reshape(B, Cin, n)
    w1f = jnp.transpose(w1, (0, 3, 2, 1)).reshape(C1, _K * _K * Cin)
    w2f = jnp.transpose(w2, (0, 3, 2, 1)).reshape(C2, _K * _K * C1)
    w1f = w1f.astype(jnp.bfloat16)
    w2f = w2f.astype(jnp.bfloat16)
    b1c = b1.reshape(C1, 1)
    b2c = b2.reshape(C2, 1)

    kern = functools.partial(_fwd_kernel, s=S, n=n, w_stride=W, keep=keep)

    out, feat_rows = pl.pallas_call(
        kern,
        grid=(B // S,),
        in_specs=[
            pl.BlockSpec((S, Cin, n), lambda b: (b, 0, 0)),
            pl.BlockSpec((C1, _K * _K * Cin), lambda b: (0, 0)),
            pl.BlockSpec((C1, 1), lambda b: (0, 0)),
            pl.BlockSpec((C2, _K * _K * C1), lambda b: (0, 0)),
            pl.BlockSpec((C2, 1), lambda b: (0, 0)),
            pl.BlockSpec((S, Nw), lambda b: (b, 0)),
        ],
        out_specs=[
            pl.BlockSpec((S, Nw), lambda b: (b, 0)),
            pl.BlockSpec((S, C2, keep), lambda b: (b, 0, 0)),
        ],
        out_shape=[
            jax.ShapeDtypeStruct((B, Nw), wav.dtype),
            jax.ShapeDtypeStruct((B, C2, keep), jnp.float32),
        ],
        compiler_params=pltpu.CompilerParams(
            dimension_semantics=("parallel",)),
    )(x_flat, w1f, b1c, w2f, b2c, wav)

    feat = feat_rows.reshape(B, C2, H2, W)[:, :, :, :W2]
    return out, feat


# speedup vs baseline: 1.5859x; 1.5859x over previous
import functools

import jax
import jax.numpy as jnp
from jax.experimental import pallas as pl
from jax.experimental.pallas import tpu as pltpu

_K = 5


def _taps(a, w_stride, length):
    rows = [a]
    for kh in range(1, _K):
        rows.append(pltpu.roll(a, length - kh * w_stride, axis=1))
    stacked = jnp.concatenate(rows, axis=0)
    cols = [stacked]
    for kw in range(1, _K):
        cols.append(pltpu.roll(stacked, length - kw, axis=1))
    return jnp.concatenate(cols, axis=0)


def _fwd_kernel(x_ref, w1_ref, b1_ref, w2_ref, b2_ref, wav_ref,
                out_ref, feat_ref, *, s, n, w_stride, keep):
    length = s * n
    xs = x_ref[...]
    xc = jnp.concatenate([xs[i] for i in range(s)], axis=1)
    xc = xc.astype(jnp.bfloat16)

    t1 = _taps(xc, w_stride, length)
    h1 = jnp.dot(w1_ref[...], t1, preferred_element_type=jnp.float32)
    h1 = jnp.maximum(h1 + b1_ref[...], 0.0).astype(jnp.bfloat16)

    t2 = _taps(h1, w_stride, length)
    h2 = jnp.dot(w2_ref[...], t2, preferred_element_type=jnp.float32)
    h2 = jnp.maximum(h2 + b2_ref[...], 0.0)

    for i in range(s):
        feat_ref[i] = h2[:, i * n:i * n + keep]

    out_ref[...] = jnp.log(wav_ref[...]) * (1.0 / 20.0)


def kernel(w1, b1, w2, b2, x, wav):
    B, Cin, H, W = x.shape
    C1, C2 = w1.shape[0], w2.shape[0]
    H2, W2 = H - 2 * (_K - 1), W - 2 * (_K - 1)
    n = H * W
    keep = H2 * W
    Nw = wav.shape[-1]
    S = 4
    assert B % S == 0

    x_flat = x.reshape(B, Cin, n)
    wav3 = wav.reshape(B // S, S, Nw)
    w1f = jnp.transpose(w1, (0, 3, 2, 1)).reshape(C1, _K * _K * Cin)
    w2f = jnp.transpose(w2, (0, 3, 2, 1)).reshape(C2, _K * _K * C1)
    w1f = w1f.astype(jnp.bfloat16)
    w2f = w2f.astype(jnp.bfloat16)
    b1c = b1.reshape(C1, 1)
    b2c = b2.reshape(C2, 1)

    kern = functools.partial(_fwd_kernel, s=S, n=n, w_stride=W, keep=keep)

    out, feat_rows = pl.pallas_call(
        kern,
        grid=(B // S,),
        in_specs=[
            pl.BlockSpec((S, Cin, n), lambda b: (b, 0, 0)),
            pl.BlockSpec((C1, _K * _K * Cin), lambda b: (0, 0)),
            pl.BlockSpec((C1, 1), lambda b: (0, 0)),
            pl.BlockSpec((C2, _K * _K * C1), lambda b: (0, 0)),
            pl.BlockSpec((C2, 1), lambda b: (0, 0)),
            pl.BlockSpec((1, S, Nw), lambda b: (b, 0, 0)),
        ],
        out_specs=[
            pl.BlockSpec((1, S, Nw), lambda b: (b, 0, 0)),
            pl.BlockSpec((S, C2, keep), lambda b: (b, 0, 0)),
        ],
        out_shape=[
            jax.ShapeDtypeStruct((B // S, S, Nw), wav.dtype),
            jax.ShapeDtypeStruct((B, C2, keep), jnp.float32),
        ],
        compiler_params=pltpu.CompilerParams(
            dimension_semantics=("parallel",)),
    )(x_flat, w1f, b1c, w2f, b2c, wav3)

    feat = feat_rows.reshape(B, C2, H2, W)[:, :, :, :W2]
    return out.reshape(B, Nw), feat


# speedup vs baseline: 1.6909x; 1.0662x over previous
import functools

import jax
import jax.numpy as jnp
from jax.experimental import pallas as pl
from jax.experimental.pallas import tpu as pltpu

_K = 5


def _row_stack(a, w_stride, length):
    rows = [a]
    for kh in range(1, _K):
        rows.append(pltpu.roll(a, length - kh * w_stride, axis=1))
    return jnp.concatenate(rows, axis=0)


def _conv(stacked, w_ref, b_ref, length):
    acc = jnp.dot(w_ref[0], stacked, preferred_element_type=jnp.float32)
    for kw in range(1, _K):
        shifted = pltpu.roll(stacked, length - kw, axis=1)
        acc = acc + jnp.dot(w_ref[kw], shifted,
                            preferred_element_type=jnp.float32)
    return jnp.maximum(acc + b_ref[...], 0.0)


def _fwd_kernel(x_ref, w1_ref, b1_ref, w2_ref, b2_ref, wav_ref,
                out_ref, feat_ref, *, s, n, w_stride, keep):
    length = s * n
    xs = x_ref[...]
    xc = jnp.concatenate([xs[i] for i in range(s)], axis=1)
    xc = xc.astype(jnp.bfloat16)

    st1 = _row_stack(xc, w_stride, length)
    h1 = _conv(st1, w1_ref, b1_ref, length)
    h1 = h1.astype(jnp.bfloat16)

    st2 = _row_stack(h1, w_stride, length)
    h2 = _conv(st2, w2_ref, b2_ref, length)

    for i in range(s):
        feat_ref[i] = h2[:, i * n:i * n + keep]

    out_ref[...] = jnp.log(wav_ref[...]) * (1.0 / 20.0)


def kernel(w1, b1, w2, b2, x, wav):
    B, Cin, H, W = x.shape
    C1, C2 = w1.shape[0], w2.shape[0]
    H2, W2 = H - 2 * (_K - 1), W - 2 * (_K - 1)
    n = H * W
    keep = H2 * W
    Nw = wav.shape[-1]
    S = 4
    assert B % S == 0

    x_flat = x.reshape(B, Cin, n)
    wav3 = wav.reshape(B // S, S, Nw)
    w1s = jnp.transpose(w1, (3, 0, 2, 1)).reshape(_K, C1, _K * Cin)
    w2s = jnp.transpose(w2, (3, 0, 2, 1)).reshape(_K, C2, _K * C1)
    w1s = w1s.astype(jnp.bfloat16)
    w2s = w2s.astype(jnp.bfloat16)
    b1c = b1.reshape(C1, 1)
    b2c = b2.reshape(C2, 1)

    kern = functools.partial(_fwd_kernel, s=S, n=n, w_stride=W, keep=keep)

    out, feat_rows = pl.pallas_call(
        kern,
        grid=(B // S,),
        in_specs=[
            pl.BlockSpec((S, Cin, n), lambda b: (b, 0, 0)),
            pl.BlockSpec((_K, C1, _K * Cin), lambda b: (0, 0, 0)),
            pl.BlockSpec((C1, 1), lambda b: (0, 0)),
            pl.BlockSpec((_K, C2, _K * C1), lambda b: (0, 0, 0)),
            pl.BlockSpec((C2, 1), lambda b: (0, 0)),
            pl.BlockSpec((1, S, Nw), lambda b: (b, 0, 0)),
        ],
        out_specs=[
            pl.BlockSpec((1, S, Nw), lambda b: (b, 0, 0)),
            pl.BlockSpec((S, C2, keep), lambda b: (b, 0, 0)),
        ],
        out_shape=[
            jax.ShapeDtypeStruct((B // S, S, Nw), wav.dtype),
            jax.ShapeDtypeStruct((B, C2, keep), jnp.float32),
        ],
        compiler_params=pltpu.CompilerParams(
            dimension_semantics=("parallel",)),
    )(x_flat, w1s, b1c, w2s, b2c, wav3)

    feat = feat_rows.reshape(B, C2, H2, W)[:, :, :, :W2]
    return out.reshape(B, Nw), feat


# speedup vs baseline: 1.7521x; 1.0362x over previous
import functools

import jax
import jax.numpy as jnp
from jax.experimental import pallas as pl
from jax.experimental.pallas import tpu as pltpu

_K = 5


def _row_stack(a, w_stride, length):
    rows = [a]
    for kh in range(1, _K):
        rows.append(pltpu.roll(a, length - kh * w_stride, axis=1))
    return jnp.concatenate(rows, axis=0)


def _conv(stacked, w_ref, b_ref, length):
    acc = jnp.dot(w_ref[0], stacked, preferred_element_type=jnp.float32)
    for kw in range(1, _K):
        shifted = pltpu.roll(stacked, length - kw, axis=1)
        acc = acc + jnp.dot(w_ref[kw], shifted,
                            preferred_element_type=jnp.float32)
    return jnp.maximum(acc + b_ref[...], 0.0)


def _fwd_kernel(x_ref, w1_ref, b1_ref, w2_ref, b2_ref, wav_ref,
                out_ref, feat_ref, *, s, n, w_stride, keep):
    length = s * n
    xs = x_ref[...]
    xc = jnp.concatenate([xs[i] for i in range(s)], axis=1)
    xc = xc.astype(jnp.bfloat16)

    st1 = _row_stack(xc, w_stride, length)
    h1 = _conv(st1, w1_ref, b1_ref, length)
    h1 = h1.astype(jnp.bfloat16)

    st2 = _row_stack(h1, w_stride, length)
    h2 = _conv(st2, w2_ref, b2_ref, length)

    for i in range(s):
        feat_ref[i] = h2[:, i * n:i * n + keep]

    out_ref[...] = jnp.log(wav_ref[...]) * (1.0 / 20.0)


def kernel(w1, b1, w2, b2, x, wav):
    B, Cin, H, W = x.shape
    C1, C2 = w1.shape[0], w2.shape[0]
    H2, W2 = H - 2 * (_K - 1), W - 2 * (_K - 1)
    n = H * W
    keep = H2 * W
    Nw = wav.shape[-1]
    S = 8
    assert B % S == 0

    x_flat = x.reshape(B, Cin, n)
    wav3 = wav.reshape(B // S, S, Nw)
    w1s = jnp.transpose(w1, (3, 0, 2, 1)).reshape(_K, C1, _K * Cin)
    w2s = jnp.transpose(w2, (3, 0, 2, 1)).reshape(_K, C2, _K * C1)
    w1s = w1s.astype(jnp.bfloat16)
    w2s = w2s.astype(jnp.bfloat16)
    b1c = b1.reshape(C1, 1)
    b2c = b2.reshape(C2, 1)

    kern = functools.partial(_fwd_kernel, s=S, n=n, w_stride=W, keep=keep)

    out, feat_rows = pl.pallas_call(
        kern,
        grid=(B // S,),
        in_specs=[
            pl.BlockSpec((S, Cin, n), lambda b: (b, 0, 0)),
            pl.BlockSpec((_K, C1, _K * Cin), lambda b: (0, 0, 0)),
            pl.BlockSpec((C1, 1), lambda b: (0, 0)),
            pl.BlockSpec((_K, C2, _K * C1), lambda b: (0, 0, 0)),
            pl.BlockSpec((C2, 1), lambda b: (0, 0)),
            pl.BlockSpec((1, S, Nw), lambda b: (b, 0, 0)),
        ],
        out_specs=[
            pl.BlockSpec((1, S, Nw), lambda b: (b, 0, 0)),
            pl.BlockSpec((S, C2, keep), lambda b: (b, 0, 0)),
        ],
        out_shape=[
            jax.ShapeDtypeStruct((B // S, S, Nw), wav.dtype),
            jax.ShapeDtypeStruct((B, C2, keep), jnp.float32),
        ],
        compiler_params=pltpu.CompilerParams(
            dimension_semantics=("parallel",)),
    )(x_flat, w1s, b1c, w2s, b2c, wav3)

    feat = feat_rows.reshape(B, C2, H2, W)[:, :, :, :W2]
    return out.reshape(B, Nw), feat
